# baseline (speedup 1.0000x reference)
import functools

import jax
import jax.numpy as jnp
from jax import lax
from jax.experimental import pallas as pl
from jax.experimental.pallas import tpu as pltpu
from jax.experimental.pallas import tpu_sc as plsc

BATCH = 16384
EMB_DIM = 32
NC = 2
NS = 16
NW = NC * NS
B_PER_W = BATCH // NW
CHUNK = 128
NCHUNK = B_PER_W // CHUNK


def _body(uid_hbm, iid_hbm, uemb_hbm, iemb_hbm, out_hbm,
          uid_v, iid_v, urows_v, irows_v, out_v, sem):
  wid = lax.axis_index("s") * NC + lax.axis_index("c")
  base = wid * B_PER_W

  for k in range(NCHUNK):
    pltpu.sync_copy(uid_hbm.at[pl.ds(base + k * CHUNK, CHUNK)], uid_v.at[k])
    pltpu.sync_copy(iid_hbm.at[pl.ds(base + k * CHUNK, CHUNK)], iid_v.at[k])

  copies = []
  for k in range(NCHUNK):
    copies.append(pltpu.async_copy(
        uemb_hbm.at[uid_v.at[k]], urows_v.at[pl.ds(k * CHUNK, CHUNK)], sem))
    copies.append(pltpu.async_copy(
        iemb_hbm.at[iid_v.at[k]], irows_v.at[pl.ds(k * CHUNK, CHUNK)], sem))
  for c in copies:
    c.wait()

  lane = jnp.arange(16, dtype=jnp.int32)

  def group(g, _):
    rid = g * 16 + lane
    acc = jnp.zeros((16,), jnp.float32)
    for d in range(EMB_DIM):
      dd = jnp.full((16,), d, jnp.int32)
      uv = plsc.load_gather(urows_v, [rid, dd])
      iv = plsc.load_gather(irows_v, [rid, dd])
      acc = acc + uv * iv
    sig = 1.0 / (1.0 + jnp.exp(-acc))
    plsc.store_scatter(out_v, [rid], sig)
    return _

  lax.fori_loop(0, B_PER_W // 16, group, None)
  pltpu.sync_copy(out_v, out_hbm.at[pl.ds(base, B_PER_W)])


@jax.jit
def _run(user_ids, item_ids, user_emb, item_emb):
  mesh = plsc.VectorSubcoreMesh(core_axis_name="c", subcore_axis_name="s")
  k = pl.kernel(
      _body,
      out_type=jax.ShapeDtypeStruct((BATCH,), jnp.float32),
      mesh=mesh,
      compiler_params=pltpu.CompilerParams(
          needs_layout_passes=False, use_tc_tiling_on_sc=False),
      scratch_types=[
          pltpu.VMEM((NCHUNK, CHUNK), jnp.int32),
          pltpu.VMEM((NCHUNK, CHUNK), jnp.int32),
          pltpu.VMEM((B_PER_W, EMB_DIM), jnp.float32),
          pltpu.VMEM((B_PER_W, EMB_DIM), jnp.float32),
          pltpu.VMEM((B_PER_W,), jnp.float32),
          pltpu.SemaphoreType.DMA,
      ],
  )
  return k(user_ids, item_ids, user_emb, item_emb)


def kernel(user_ids, item_ids, user_emb, item_emb):
  return _run(user_ids.astype(jnp.int32), item_ids.astype(jnp.int32),
              user_emb, item_emb)
